# Initial kernel scaffold; baseline (speedup 1.0000x reference)
import jax
import jax.numpy as jnp
from jax.experimental import pallas as pl


def kernel(x):
    raise NotImplementedError("write your pallas kernel here")


# speedup vs baseline: 141.4108x; 141.4108x over previous
import jax
import jax.numpy as jnp
from jax import lax
from jax.experimental import pallas as pl

OMEGA_K = 256
NBITS = 31


def _thr_body(x_ref, o_ref):
    xb = x_ref[...]
    bits = lax.bitcast_convert_type(jnp.abs(xb), jnp.int32)

    def step(i, t):
        cand = t | (1 << (30 - i))
        cnt = jnp.sum((bits >= cand[None, :]).astype(jnp.int32), axis=0)
        return jnp.where(cnt >= OMEGA_K, cand, t)

    t0 = jnp.zeros((128,), jnp.int32)
    thr = lax.fori_loop(0, NBITS, step, t0)
    o_ref[...] = jnp.where(bits >= thr[None, :], xb, 0.0)


def kernel(x):
    b, w, d = x.shape
    return pl.pallas_call(
        _thr_body,
        grid=(b,),
        in_specs=[pl.BlockSpec((None, w, d), lambda i: (i, 0, 0))],
        out_specs=pl.BlockSpec((None, w, d), lambda i: (i, 0, 0)),
        out_shape=jax.ShapeDtypeStruct(x.shape, x.dtype),
    )(x)


# speedup vs baseline: 284.4465x; 2.0115x over previous
import jax
import jax.numpy as jnp
from jax import lax
from jax.experimental import pallas as pl

OMEGA_K = 256
NBITS = 31


def _thr_body(x_ref, o_ref):
    xb = x_ref[...]
    bits = lax.bitcast_convert_type(jnp.abs(xb), jnp.int32)
    ones = jnp.ones((8, xb.shape[0]), jnp.float32)

    def step(i, t):
        cand = t | (1 << (30 - i))
        maskf = jnp.where(bits >= cand[None, :], 1.0, 0.0)
        cnt = jnp.dot(ones, maskf, preferred_element_type=jnp.float32)[0]
        return jnp.where(cnt >= float(OMEGA_K), cand, t)

    t0 = jnp.zeros((128,), jnp.int32)
    thr = lax.fori_loop(0, NBITS, step, t0)
    o_ref[...] = jnp.where(bits >= thr[None, :], xb, 0.0)


def kernel(x):
    b, w, d = x.shape
    return pl.pallas_call(
        _thr_body,
        grid=(b,),
        in_specs=[pl.BlockSpec((None, w, d), lambda i: (i, 0, 0))],
        out_specs=pl.BlockSpec((None, w, d), lambda i: (i, 0, 0)),
        out_shape=jax.ShapeDtypeStruct(x.shape, x.dtype),
    )(x)
